# Initial kernel scaffold; baseline (speedup 1.0000x reference)
import functools

import jax
import jax.numpy as jnp
from jax import lax
from jax.experimental import pallas as pl
from jax.experimental.pallas import tpu as pltpu
from jax.experimental.pallas import tpu_sc as plsc

N_NODES = 10000
N_PAD = 10016
E_EDGES = 320000
NUM_WORKERS = 32
CHUNKS = 80
CHUNK = 128
E_PAD = NUM_WORKERS * CHUNKS * CHUNK
D_IN = 128
D_HID = 32
D_CLS = 10
D_CLS_PAD = 16
ZROWS = N_PAD // 16

_mesh = plsc.VectorSubcoreMesh(core_axis_name="c", subcore_axis_name="s")


@functools.partial(
    pl.kernel,
    mesh=_mesh,
    out_type=jax.ShapeDtypeStruct((2, N_PAD, 16), jnp.float32),
    scratch_types=[
        pltpu.VMEM((CHUNKS, CHUNK), jnp.int32),
        pltpu.VMEM((CHUNK, 16), jnp.float32),
        pltpu.VMEM((ZROWS, 16), jnp.float32),
        pltpu.VMEM_SHARED((N_PAD, 16), jnp.float32),
    ],
)
def _sc_degree(dst_hbm, out_hbm, idx_v, ones_v, zero_v, acc_sh):
    c = lax.axis_index("c")
    s = lax.axis_index("s")
    wid = c * 16 + s

    @pl.loop(0, CHUNK)
    def _(r):
        ones_v[r, :] = jnp.full((16,), 1.0, jnp.float32)

    @pl.loop(0, ZROWS)
    def _(r):
        zero_v[r, :] = jnp.zeros((16,), jnp.float32)

    pltpu.sync_copy(zero_v, acc_sh.at[pl.ds(s * ZROWS, ZROWS)])
    pltpu.sync_copy(dst_hbm.at[wid], idx_v)
    plsc.subcore_barrier()

    @pl.loop(0, CHUNKS)
    def _(j):
        pltpu.sync_copy(ones_v, acc_sh.at[idx_v.at[j]], add=True)

    plsc.subcore_barrier()
    pltpu.sync_copy(acc_sh.at[pl.ds(s * ZROWS, ZROWS)],
                    out_hbm.at[c, pl.ds(s * ZROWS, ZROWS)])


def _make_sc_agg(depth):
    @functools.partial(
        pl.kernel,
        mesh=_mesh,
        out_type=jax.ShapeDtypeStruct((2, N_PAD, depth), jnp.float32),
        scratch_types=[
            pltpu.VMEM((CHUNKS, CHUNK), jnp.int32),
            pltpu.VMEM((CHUNKS, CHUNK), jnp.int32),
            pltpu.VMEM((CHUNK, depth), jnp.float32),
            pltpu.VMEM((CHUNK, depth), jnp.float32),
            pltpu.VMEM((ZROWS, depth), jnp.float32),
            pltpu.VMEM_SHARED((N_PAD, depth), jnp.float32),
            pltpu.SemaphoreType.DMA,
            pltpu.SemaphoreType.DMA,
        ],
    )
    def _agg(g_hbm, src_hbm, dst_hbm, out_hbm,
             src_v, dst_v, buf_a, buf_b, zero_v, acc_sh, sem_a, sem_b):
        c = lax.axis_index("c")
        s = lax.axis_index("s")
        wid = c * 16 + s

        @pl.loop(0, ZROWS)
        def _(r):
            for t in range(depth // 16):
                zero_v[r, pl.ds(t * 16, 16)] = jnp.zeros((16,), jnp.float32)

        pltpu.sync_copy(zero_v, acc_sh.at[pl.ds(s * ZROWS, ZROWS)])
        pltpu.sync_copy(src_hbm.at[wid], src_v)
        pltpu.sync_copy(dst_hbm.at[wid], dst_v)
        plsc.subcore_barrier()

        @pl.loop(0, CHUNKS, step=2)
        def _(j):
            ca = pltpu.async_copy(g_hbm.at[src_v.at[j]], buf_a, sem_a)
            cb = pltpu.async_copy(g_hbm.at[src_v.at[j + 1]], buf_b, sem_b)
            ca.wait()
            pltpu.sync_copy(buf_a, acc_sh.at[dst_v.at[j]], add=True)
            cb.wait()
            pltpu.sync_copy(buf_b, acc_sh.at[dst_v.at[j + 1]], add=True)

        plsc.subcore_barrier()
        pltpu.sync_copy(acc_sh.at[pl.ds(s * ZROWS, ZROWS)],
                        out_hbm.at[c, pl.ds(s * ZROWS, ZROWS)])

    return _agg


_sc_agg32 = _make_sc_agg(D_HID)
_sc_agg16 = _make_sc_agg(D_CLS_PAD)


def _tc_mm1(x, w1):
    def body(x_ref, w_ref, o_ref):
        o_ref[...] = jnp.dot(x_ref[...], w_ref[...],
                             preferred_element_type=jnp.float32)

    return pl.pallas_call(
        body,
        out_shape=jax.ShapeDtypeStruct((N_PAD, D_HID), jnp.float32),
    )(x, w1)


def _tc_scale(degp, h1):
    def body(d_ref, h_ref, dis_ref, g_ref):
        d = d_ref[...]
        deg = d[0, :, 0:1] + d[1, :, 0:1] + 1.0
        dis = lax.rsqrt(deg)
        dis_ref[...] = dis
        g_ref[...] = h_ref[...] * dis

    return pl.pallas_call(
        body,
        out_shape=[
            jax.ShapeDtypeStruct((N_PAD, 1), jnp.float32),
            jax.ShapeDtypeStruct((N_PAD, D_HID), jnp.float32),
        ],
    )(degp, h1)


def _tc_layer2(aggp, g1, dis, b1r, w2p):
    def body(a_ref, g_ref, dis_ref, b_ref, w_ref, o_ref):
        a = a_ref[...]
        dis = dis_ref[...]
        u = (a[0] + a[1] + g_ref[...]) * dis + b_ref[...]
        t = jnp.tanh(u)
        h2 = jnp.dot(t, w_ref[...], preferred_element_type=jnp.float32)
        o_ref[...] = h2 * dis

    return pl.pallas_call(
        body,
        out_shape=jax.ShapeDtypeStruct((N_PAD, D_CLS_PAD), jnp.float32),
    )(aggp, g1, dis, b1r, w2p)


def _tc_out(aggp, g2, dis, b2r):
    def body(a_ref, g_ref, dis_ref, b_ref, o_ref):
        a = a_ref[...]
        u = (a[0] + a[1] + g_ref[...]) * dis_ref[...] + b_ref[...]
        logits = u[:N_NODES, :D_CLS]
        m = jnp.max(logits, axis=1, keepdims=True)
        sh = logits - m
        lse = jnp.log(jnp.sum(jnp.exp(sh), axis=1, keepdims=True))
        o_ref[...] = sh - lse

    return pl.pallas_call(
        body,
        out_shape=jax.ShapeDtypeStruct((N_NODES, D_CLS), jnp.float32),
    )(aggp, g2, dis, b2r)


def kernel(x, edge_index, W1, b1, W2, b2):
    src = edge_index[0]
    dst = edge_index[1]
    pad_idx = jnp.full((E_PAD - E_EDGES,), N_NODES, dtype=jnp.int32)
    src_r = jnp.concatenate([src, pad_idx]).reshape(NUM_WORKERS, CHUNKS, CHUNK)
    dst_r = jnp.concatenate([dst, pad_idx]).reshape(NUM_WORKERS, CHUNKS, CHUNK)
    x_pad = jnp.pad(x, ((0, N_PAD - N_NODES), (0, 0)))
    w2p = jnp.pad(W2, ((0, 0), (0, D_CLS_PAD - D_CLS)))
    b1r = b1.reshape(1, D_HID)
    b2r = jnp.pad(b2, (0, D_CLS_PAD - D_CLS)).reshape(1, D_CLS_PAD)

    h1 = _tc_mm1(x_pad, W1)
    degp = _sc_degree(dst_r)
    dis, g1 = _tc_scale(degp, h1)
    agg1 = _sc_agg32(g1, src_r, dst_r)
    g2 = _tc_layer2(agg1, g1, dis, b1r, w2p)
    agg2 = _sc_agg16(g2, src_r, dst_r)
    return _tc_out(agg2, g2, dis, b2r)


# speedup vs baseline: 26.7704x; 26.7704x over previous
import functools

import jax
import jax.numpy as jnp
from jax import lax
from jax.experimental import pallas as pl
from jax.experimental.pallas import tpu as pltpu
from jax.experimental.pallas import tpu_sc as plsc

N_NODES = 10000
N_PAD = 10112
E_EDGES = 320000
NUM_WORKERS = 32
CHUNKS = 80
CHUNK = 128
E_PAD = NUM_WORKERS * CHUNKS * CHUNK
D_IN = 128
D_HID = 32
D_CLS = 10
D_CLS_PAD = 16
ZROWS = N_PAD // 16

_mesh = plsc.VectorSubcoreMesh(core_axis_name="c", subcore_axis_name="s")
_sc_params = pltpu.CompilerParams(use_tc_tiling_on_sc=False)


@functools.partial(
    pl.kernel,
    mesh=_mesh,
    compiler_params=_sc_params,
    out_type=jax.ShapeDtypeStruct((2, N_PAD, 16), jnp.float32),
    scratch_types=[
        pltpu.VMEM((CHUNKS, CHUNK), jnp.int32),
        pltpu.VMEM((CHUNK, 16), jnp.float32),
        pltpu.VMEM((ZROWS, 16), jnp.float32),
        pltpu.VMEM_SHARED((N_PAD, 16), jnp.float32),
    ],
)
def _sc_degree(dst_hbm, out_hbm, idx_v, ones_v, zero_v, acc_sh):
    c = lax.axis_index("c")
    s = lax.axis_index("s")
    wid = c * 16 + s

    @pl.loop(0, CHUNK)
    def _(r):
        ones_v[r, :] = jnp.full((16,), 1.0, jnp.float32)

    @pl.loop(0, ZROWS)
    def _(r):
        zero_v[r, :] = jnp.zeros((16,), jnp.float32)

    pltpu.sync_copy(zero_v, acc_sh.at[pl.ds(s * ZROWS, ZROWS)])
    pltpu.sync_copy(dst_hbm.at[wid], idx_v)
    plsc.subcore_barrier()

    @pl.loop(0, CHUNKS)
    def _(j):
        pltpu.sync_copy(ones_v, acc_sh.at[idx_v.at[j]], add=True)

    plsc.subcore_barrier()
    pltpu.sync_copy(acc_sh.at[pl.ds(s * ZROWS, ZROWS)],
                    out_hbm.at[c, pl.ds(s * ZROWS, ZROWS)])


def _make_sc_agg(depth):
    @functools.partial(
        pl.kernel,
        mesh=_mesh,
        compiler_params=_sc_params,
        out_type=jax.ShapeDtypeStruct((2, N_PAD, depth), jnp.float32),
        scratch_types=[
            pltpu.VMEM((CHUNKS, CHUNK), jnp.int32),
            pltpu.VMEM((CHUNKS, CHUNK), jnp.int32),
            pltpu.VMEM((CHUNK, depth), jnp.float32),
            pltpu.VMEM((CHUNK, depth), jnp.float32),
            pltpu.VMEM((ZROWS, depth), jnp.float32),
            pltpu.VMEM_SHARED((N_PAD, depth), jnp.float32),
            pltpu.SemaphoreType.DMA,
            pltpu.SemaphoreType.DMA,
        ],
    )
    def _agg(g_hbm, src_hbm, dst_hbm, out_hbm,
             src_v, dst_v, buf_a, buf_b, zero_v, acc_sh, sem_a, sem_b):
        c = lax.axis_index("c")
        s = lax.axis_index("s")
        wid = c * 16 + s

        @pl.loop(0, ZROWS)
        def _(r):
            for t in range(depth // 16):
                zero_v[r, pl.ds(t * 16, 16)] = jnp.zeros((16,), jnp.float32)

        pltpu.sync_copy(zero_v, acc_sh.at[pl.ds(s * ZROWS, ZROWS)])
        pltpu.sync_copy(src_hbm.at[wid], src_v)
        pltpu.sync_copy(dst_hbm.at[wid], dst_v)
        plsc.subcore_barrier()

        @pl.loop(0, CHUNKS, step=2)
        def _(j):
            ca = pltpu.async_copy(g_hbm.at[src_v.at[j]], buf_a, sem_a)
            cb = pltpu.async_copy(g_hbm.at[src_v.at[j + 1]], buf_b, sem_b)
            ca.wait()
            pltpu.sync_copy(buf_a, acc_sh.at[dst_v.at[j]], add=True)
            cb.wait()
            pltpu.sync_copy(buf_b, acc_sh.at[dst_v.at[j + 1]], add=True)

        plsc.subcore_barrier()
        pltpu.sync_copy(acc_sh.at[pl.ds(s * ZROWS, ZROWS)],
                        out_hbm.at[c, pl.ds(s * ZROWS, ZROWS)])

    return _agg


_sc_agg32 = _make_sc_agg(D_HID)
_sc_agg16 = _make_sc_agg(D_CLS_PAD)


def _tc_mm1(x, w1):
    def body(x_ref, w_ref, o_ref):
        o_ref[...] = jnp.dot(x_ref[...], w_ref[...],
                             preferred_element_type=jnp.float32)

    return pl.pallas_call(
        body,
        out_shape=jax.ShapeDtypeStruct((N_PAD, D_HID), jnp.float32),
    )(x, w1)


def _tc_scale(degp, h1):
    def body(d_ref, h_ref, dis_ref, g_ref):
        d = d_ref[...]
        deg = d[0, :, 0:1] + d[1, :, 0:1] + 1.0
        dis = lax.rsqrt(deg)
        dis_ref[...] = dis
        g_ref[...] = h_ref[...] * dis

    return pl.pallas_call(
        body,
        out_shape=[
            jax.ShapeDtypeStruct((N_PAD, 1), jnp.float32),
            jax.ShapeDtypeStruct((N_PAD, D_HID), jnp.float32),
        ],
    )(degp, h1)


def _tc_layer2(aggp, g1, dis, b1r, w2p):
    def body(a_ref, g_ref, dis_ref, b_ref, w_ref, o_ref):
        a = a_ref[...]
        dis = dis_ref[...]
        u = (a[0] + a[1] + g_ref[...]) * dis + b_ref[...]
        t = jnp.tanh(u)
        h2 = jnp.dot(t, w_ref[...], preferred_element_type=jnp.float32)
        o_ref[...] = h2 * dis

    return pl.pallas_call(
        body,
        out_shape=jax.ShapeDtypeStruct((N_PAD, D_CLS_PAD), jnp.float32),
    )(aggp, g1, dis, b1r, w2p)


def _tc_out(aggp, g2, dis, b2r):
    def body(a_ref, g_ref, dis_ref, b_ref, o_ref):
        a = a_ref[...]
        u = (a[0] + a[1] + g_ref[...]) * dis_ref[...] + b_ref[...]
        logits = u[:N_NODES, :D_CLS]
        m = jnp.max(logits, axis=1, keepdims=True)
        sh = logits - m
        lse = jnp.log(jnp.sum(jnp.exp(sh), axis=1, keepdims=True))
        o_ref[...] = sh - lse

    return pl.pallas_call(
        body,
        out_shape=jax.ShapeDtypeStruct((N_NODES, D_CLS), jnp.float32),
    )(aggp, g2, dis, b2r)


def kernel(x, edge_index, W1, b1, W2, b2):
    src = edge_index[0]
    dst = edge_index[1]
    pad_idx = jnp.full((E_PAD - E_EDGES,), N_NODES, dtype=jnp.int32)
    src_r = jnp.concatenate([src, pad_idx]).reshape(NUM_WORKERS, CHUNKS, CHUNK)
    dst_r = jnp.concatenate([dst, pad_idx]).reshape(NUM_WORKERS, CHUNKS, CHUNK)
    x_pad = jnp.pad(x, ((0, N_PAD - N_NODES), (0, 0)))
    w2p = jnp.pad(W2, ((0, 0), (0, D_CLS_PAD - D_CLS)))
    b1r = b1.reshape(1, D_HID)
    b2r = jnp.pad(b2, (0, D_CLS_PAD - D_CLS)).reshape(1, D_CLS_PAD)

    h1 = _tc_mm1(x_pad, W1)
    degp = _sc_degree(dst_r)
    dis, g1 = _tc_scale(degp, h1)
    agg1 = _sc_agg32(g1, src_r, dst_r)
    g2 = _tc_layer2(agg1, g1, dis, b1r, w2p)
    agg2 = _sc_agg16(g2, src_r, dst_r)
    return _tc_out(agg2, g2, dis, b2r)


# speedup vs baseline: 29.6437x; 1.1073x over previous
import functools

import jax
import jax.numpy as jnp
from jax import lax
from jax.experimental import pallas as pl
from jax.experimental.pallas import tpu as pltpu
from jax.experimental.pallas import tpu_sc as plsc

N_NODES = 10000
N_PAD = 10112
E_EDGES = 320000
NUM_WORKERS = 32
CHUNKS = 80
CHUNK = 128
E_PAD = NUM_WORKERS * CHUNKS * CHUNK
D_IN = 128
D_HID = 32
D_CLS = 10
D_CLS_PAD = 16
ZROWS = N_PAD // 16

_mesh = plsc.VectorSubcoreMesh(core_axis_name="c", subcore_axis_name="s")
_sc_params = pltpu.CompilerParams(use_tc_tiling_on_sc=False)


@functools.partial(
    pl.kernel,
    mesh=_mesh,
    compiler_params=_sc_params,
    out_type=jax.ShapeDtypeStruct((2, N_PAD, 16), jnp.float32),
    scratch_types=[
        pltpu.VMEM((CHUNKS, CHUNK), jnp.int32),
        pltpu.VMEM((CHUNK, 16), jnp.float32),
        pltpu.VMEM((ZROWS, 16), jnp.float32),
        pltpu.VMEM_SHARED((N_PAD, 16), jnp.float32),
        pltpu.SemaphoreType.DMA,
    ],
)
def _sc_degree(dst_hbm, out_hbm, idx_v, ones_v, zero_v, acc_sh, sem):
    c = lax.axis_index("c")
    s = lax.axis_index("s")
    wid = c * 16 + s

    @pl.loop(0, CHUNK)
    def _(r):
        ones_v[r, :] = jnp.full((16,), 1.0, jnp.float32)

    @pl.loop(0, ZROWS)
    def _(r):
        zero_v[r, :] = jnp.zeros((16,), jnp.float32)

    pltpu.sync_copy(zero_v, acc_sh.at[pl.ds(s * ZROWS, ZROWS)])
    pltpu.sync_copy(dst_hbm.at[wid], idx_v)
    plsc.subcore_barrier()

    DEPTH = 8
    for j in range(DEPTH):
        pltpu.async_copy(ones_v, acc_sh.at[idx_v.at[j]], sem, add=True)

    @pl.loop(DEPTH, CHUNKS)
    def _(j):
        pltpu.make_async_copy(ones_v, acc_sh.at[idx_v.at[j]], sem).wait()
        pltpu.async_copy(ones_v, acc_sh.at[idx_v.at[j]], sem, add=True)

    for j in range(DEPTH):
        pltpu.make_async_copy(ones_v, acc_sh.at[idx_v.at[j]], sem).wait()

    plsc.subcore_barrier()
    pltpu.sync_copy(acc_sh.at[pl.ds(s * ZROWS, ZROWS)],
                    out_hbm.at[c, pl.ds(s * ZROWS, ZROWS)])


def _make_sc_agg(depth):
    @functools.partial(
        pl.kernel,
        mesh=_mesh,
        compiler_params=_sc_params,
        out_type=jax.ShapeDtypeStruct((2, N_PAD, depth), jnp.float32),
        scratch_types=[
            pltpu.VMEM((CHUNKS, CHUNK), jnp.int32),
            pltpu.VMEM((CHUNKS, CHUNK), jnp.int32),
            pltpu.VMEM((4, CHUNK, depth), jnp.float32),
            pltpu.VMEM((ZROWS, depth), jnp.float32),
            pltpu.VMEM_SHARED((N_PAD, depth), jnp.float32),
            pltpu.SemaphoreType.DMA,
            pltpu.SemaphoreType.DMA,
            pltpu.SemaphoreType.DMA,
            pltpu.SemaphoreType.DMA,
            pltpu.SemaphoreType.DMA,
            pltpu.SemaphoreType.DMA,
            pltpu.SemaphoreType.DMA,
            pltpu.SemaphoreType.DMA,
        ],
    )
    def _agg(g_hbm, src_hbm, dst_hbm, out_hbm,
             src_v, dst_v, bufs, zero_v, acc_sh,
             g0, g1, g2, g3, s0, s1, s2, s3):
        c = lax.axis_index("c")
        s = lax.axis_index("s")
        wid = c * 16 + s
        gsem = (g0, g1, g2, g3)
        ssem = (s0, s1, s2, s3)

        @pl.loop(0, ZROWS)
        def _(r):
            for t in range(depth // 16):
                zero_v[r, pl.ds(t * 16, 16)] = jnp.zeros((16,), jnp.float32)

        pltpu.sync_copy(zero_v, acc_sh.at[pl.ds(s * ZROWS, ZROWS)])
        pltpu.sync_copy(src_hbm.at[wid], src_v)
        pltpu.sync_copy(dst_hbm.at[wid], dst_v)

        for b in range(4):
            pltpu.async_copy(g_hbm.at[src_v.at[b]], bufs.at[b], gsem[b])
        plsc.subcore_barrier()

        @pl.loop(0, CHUNKS, step=4)
        def _(j):
            for b in range(4):
                pltpu.make_async_copy(
                    g_hbm.at[src_v.at[j + b]], bufs.at[b], gsem[b]).wait()
                pltpu.async_copy(
                    bufs.at[b], acc_sh.at[dst_v.at[j + b]], ssem[b], add=True)
            for b in range(4):
                pltpu.make_async_copy(
                    bufs.at[b], acc_sh.at[dst_v.at[j + b]], ssem[b]).wait()

                @pl.when(j + 4 + b < CHUNKS)
                def _():
                    pltpu.async_copy(
                        g_hbm.at[src_v.at[j + 4 + b]], bufs.at[b], gsem[b])

        plsc.subcore_barrier()
        pltpu.sync_copy(acc_sh.at[pl.ds(s * ZROWS, ZROWS)],
                        out_hbm.at[c, pl.ds(s * ZROWS, ZROWS)])

    return _agg


_sc_agg32 = _make_sc_agg(D_HID)
_sc_agg16 = _make_sc_agg(D_CLS_PAD)


def _tc_mm1(x, w1):
    def body(x_ref, w_ref, o_ref):
        o_ref[...] = jnp.dot(x_ref[...], w_ref[...],
                             preferred_element_type=jnp.float32)

    return pl.pallas_call(
        body,
        out_shape=jax.ShapeDtypeStruct((N_PAD, D_HID), jnp.float32),
    )(x, w1)


def _tc_scale(degp, h1):
    def body(d_ref, h_ref, dis_ref, g_ref):
        d = d_ref[...]
        deg = d[0, :, 0:1] + d[1, :, 0:1] + 1.0
        dis = lax.rsqrt(deg)
        dis_ref[...] = dis
        g_ref[...] = h_ref[...] * dis

    return pl.pallas_call(
        body,
        out_shape=[
            jax.ShapeDtypeStruct((N_PAD, 1), jnp.float32),
            jax.ShapeDtypeStruct((N_PAD, D_HID), jnp.float32),
        ],
    )(degp, h1)


def _tc_layer2(aggp, g1, dis, b1r, w2p):
    def body(a_ref, g_ref, dis_ref, b_ref, w_ref, o_ref):
        a = a_ref[...]
        dis = dis_ref[...]
        u = (a[0] + a[1] + g_ref[...]) * dis + b_ref[...]
        t = jnp.tanh(u)
        h2 = jnp.dot(t, w_ref[...], preferred_element_type=jnp.float32)
        o_ref[...] = h2 * dis

    return pl.pallas_call(
        body,
        out_shape=jax.ShapeDtypeStruct((N_PAD, D_CLS_PAD), jnp.float32),
    )(aggp, g1, dis, b1r, w2p)


def _tc_out(aggp, g2, dis, b2r):
    def body(a_ref, g_ref, dis_ref, b_ref, o_ref):
        a = a_ref[...]
        u = (a[0] + a[1] + g_ref[...]) * dis_ref[...] + b_ref[...]
        logits = u[:N_NODES, :D_CLS]
        m = jnp.max(logits, axis=1, keepdims=True)
        sh = logits - m
        lse = jnp.log(jnp.sum(jnp.exp(sh), axis=1, keepdims=True))
        o_ref[...] = sh - lse

    return pl.pallas_call(
        body,
        out_shape=jax.ShapeDtypeStruct((N_NODES, D_CLS), jnp.float32),
    )(aggp, g2, dis, b2r)


def kernel(x, edge_index, W1, b1, W2, b2):
    src = edge_index[0]
    dst = edge_index[1]
    pad_idx = jnp.full((E_PAD - E_EDGES,), N_NODES, dtype=jnp.int32)
    src_r = jnp.concatenate([src, pad_idx]).reshape(NUM_WORKERS, CHUNKS, CHUNK)
    dst_r = jnp.concatenate([dst, pad_idx]).reshape(NUM_WORKERS, CHUNKS, CHUNK)
    x_pad = jnp.pad(x, ((0, N_PAD - N_NODES), (0, 0)))
    w2p = jnp.pad(W2, ((0, 0), (0, D_CLS_PAD - D_CLS)))
    b1r = b1.reshape(1, D_HID)
    b2r = jnp.pad(b2, (0, D_CLS_PAD - D_CLS)).reshape(1, D_CLS_PAD)

    h1 = _tc_mm1(x_pad, W1)
    degp = _sc_degree(dst_r)
    dis, g1 = _tc_scale(degp, h1)
    agg1 = _sc_agg32(g1, src_r, dst_r)
    g2 = _tc_layer2(agg1, g1, dis, b1r, w2p)
    agg2 = _sc_agg16(g2, src_r, dst_r)
    return _tc_out(agg2, g2, dis, b2r)


# speedup vs baseline: 51.5034x; 1.7374x over previous
import functools

import jax
import jax.numpy as jnp
from jax import lax
from jax.experimental import pallas as pl
from jax.experimental.pallas import tpu as pltpu
from jax.experimental.pallas import tpu_sc as plsc

N_NODES = 10000
N_PAD = 10112
E_EDGES = 320000
NUM_WORKERS = 32
CHUNKS = 80
CHUNK = 128
E_PAD = NUM_WORKERS * CHUNKS * CHUNK
D_IN = 128
D_HID = 32
D_CLS = 10
D_CLS_PAD = 16
ZROWS = N_PAD // 16

_mesh = plsc.VectorSubcoreMesh(core_axis_name="c", subcore_axis_name="s")
_sc_params = pltpu.CompilerParams(use_tc_tiling_on_sc=False)


@functools.partial(
    pl.kernel,
    mesh=_mesh,
    compiler_params=_sc_params,
    out_type=jax.ShapeDtypeStruct((2, N_PAD, 16), jnp.float32),
    scratch_types=[
        pltpu.VMEM((CHUNKS, CHUNK), jnp.int32),
        pltpu.VMEM((CHUNK, 16), jnp.float32),
        pltpu.VMEM((ZROWS, 16), jnp.float32),
        pltpu.VMEM_SHARED((N_PAD, 16), jnp.float32),
        pltpu.SemaphoreType.DMA,
    ],
)
def _sc_degree(dst_hbm, out_hbm, idx_v, ones_v, zero_v, acc_sh, sem):
    c = lax.axis_index("c")
    s = lax.axis_index("s")
    wid = c * 16 + s

    @pl.loop(0, CHUNK)
    def _(r):
        ones_v[r, :] = jnp.full((16,), 1.0, jnp.float32)

    @pl.loop(0, ZROWS)
    def _(r):
        zero_v[r, :] = jnp.zeros((16,), jnp.float32)

    pltpu.sync_copy(zero_v, acc_sh.at[pl.ds(s * ZROWS, ZROWS)])
    pltpu.sync_copy(dst_hbm.at[wid], idx_v)
    plsc.subcore_barrier()

    DEPTH = 8
    for j in range(DEPTH):
        pltpu.async_copy(ones_v, acc_sh.at[idx_v.at[j]], sem, add=True)

    @pl.loop(DEPTH, CHUNKS)
    def _(j):
        pltpu.make_async_copy(ones_v, acc_sh.at[idx_v.at[j]], sem).wait()
        pltpu.async_copy(ones_v, acc_sh.at[idx_v.at[j]], sem, add=True)

    for j in range(DEPTH):
        pltpu.make_async_copy(ones_v, acc_sh.at[idx_v.at[j]], sem).wait()

    plsc.subcore_barrier()
    pltpu.sync_copy(acc_sh.at[pl.ds(s * ZROWS, ZROWS)],
                    out_hbm.at[c, pl.ds(s * ZROWS, ZROWS)])


def _make_sc_agg(depth):
    @functools.partial(
        pl.kernel,
        mesh=_mesh,
        compiler_params=_sc_params,
        out_type=jax.ShapeDtypeStruct((2, N_PAD, depth), jnp.float32),
        scratch_types=[
            pltpu.VMEM((CHUNKS, CHUNK), jnp.int32),
            pltpu.VMEM((CHUNKS, CHUNK), jnp.int32),
            pltpu.VMEM((4, CHUNK, depth), jnp.float32),
            pltpu.VMEM((ZROWS, depth), jnp.float32),
            pltpu.VMEM_SHARED((N_PAD, depth), jnp.float32),
            pltpu.SemaphoreType.DMA,
            pltpu.SemaphoreType.DMA,
            pltpu.SemaphoreType.DMA,
            pltpu.SemaphoreType.DMA,
            pltpu.SemaphoreType.DMA,
            pltpu.SemaphoreType.DMA,
            pltpu.SemaphoreType.DMA,
            pltpu.SemaphoreType.DMA,
        ],
    )
    def _agg(g_hbm, src_hbm, dst_hbm, out_hbm,
             src_v, dst_v, bufs, zero_v, acc_sh,
             g0, g1, g2, g3, s0, s1, s2, s3):
        c = lax.axis_index("c")
        s = lax.axis_index("s")
        wid = c * 16 + s
        gsem = (g0, g1, g2, g3)
        ssem = (s0, s1, s2, s3)

        @pl.loop(0, ZROWS)
        def _(r):
            for t in range(depth // 16):
                zero_v[r, pl.ds(t * 16, 16)] = jnp.zeros((16,), jnp.float32)

        pltpu.sync_copy(zero_v, acc_sh.at[pl.ds(s * ZROWS, ZROWS)])
        pltpu.sync_copy(src_hbm.at[wid], src_v)
        pltpu.sync_copy(dst_hbm.at[wid], dst_v)

        for b in range(4):
            pltpu.async_copy(g_hbm.at[src_v.at[b]], bufs.at[b], gsem[b])
        plsc.subcore_barrier()

        @pl.loop(0, CHUNKS, step=4)
        def _(j):
            for b in range(4):
                pltpu.make_async_copy(
                    g_hbm.at[src_v.at[j + b]], bufs.at[b], gsem[b]).wait()
                pltpu.async_copy(
                    bufs.at[b], acc_sh.at[dst_v.at[j + b]], ssem[b], add=True)
            for b in range(4):
                pltpu.make_async_copy(
                    bufs.at[b], acc_sh.at[dst_v.at[j + b]], ssem[b]).wait()

                @pl.when(j + 4 + b < CHUNKS)
                def _():
                    pltpu.async_copy(
                        g_hbm.at[src_v.at[j + 4 + b]], bufs.at[b], gsem[b])

        plsc.subcore_barrier()
        pltpu.sync_copy(acc_sh.at[pl.ds(s * ZROWS, ZROWS)],
                        out_hbm.at[c, pl.ds(s * ZROWS, ZROWS)])

    return _agg


_sc_agg32 = _make_sc_agg(D_HID)
_sc_agg16 = _make_sc_agg(D_CLS_PAD)


def _tc_mm1(x, w1):
    def body(x_ref, w_ref, o_ref):
        o_ref[...] = jnp.dot(x_ref[...], w_ref[...],
                             preferred_element_type=jnp.float32)

    return pl.pallas_call(
        body,
        out_shape=jax.ShapeDtypeStruct((N_PAD, D_HID), jnp.float32),
    )(x, w1)


def _tc_scale(degp, h1):
    def body(d_ref, h_ref, dis_ref, g_ref):
        d = d_ref[...]
        deg = d[0, :, 0:1] + d[1, :, 0:1] + 1.0
        dis = lax.rsqrt(deg)
        dis_ref[...] = dis
        g_ref[...] = h_ref[...] * dis

    return pl.pallas_call(
        body,
        out_shape=[
            jax.ShapeDtypeStruct((N_PAD, 1), jnp.float32),
            jax.ShapeDtypeStruct((N_PAD, D_HID), jnp.float32),
        ],
    )(degp, h1)


def _tc_layer2(aggp, g1, dis, b1r, w2p):
    def body(a_ref, g_ref, dis_ref, b_ref, w_ref, o_ref):
        a = a_ref[...]
        dis = dis_ref[...]
        u = (a[0] + a[1] + g_ref[...]) * dis + b_ref[...]
        t = jnp.tanh(u)
        h2 = jnp.dot(t, w_ref[...], preferred_element_type=jnp.float32)
        o_ref[...] = h2 * dis

    return pl.pallas_call(
        body,
        out_shape=jax.ShapeDtypeStruct((N_PAD, D_CLS_PAD), jnp.float32),
    )(aggp, g1, dis, b1r, w2p)


def _tc_out(aggp, g2, dis, b2r):
    def body(a_ref, g_ref, dis_ref, b_ref, o_ref):
        a = a_ref[...]
        u = (a[0] + a[1] + g_ref[...]) * dis_ref[...] + b_ref[...]
        logits = u[:N_NODES, :D_CLS]
        m = jnp.max(logits, axis=1, keepdims=True)
        sh = logits - m
        lse = jnp.log(jnp.sum(jnp.exp(sh), axis=1, keepdims=True))
        o_ref[...] = sh - lse

    return pl.pallas_call(
        body,
        out_shape=jax.ShapeDtypeStruct((N_NODES, D_CLS), jnp.float32),
    )(aggp, g2, dis, b2r)


def kernel(x, edge_index, W1, b1, W2, b2):
    src = edge_index[0]
    dst = edge_index[1]
    pad_idx = (N_NODES +
               jnp.arange(E_PAD - E_EDGES, dtype=jnp.int32) % (N_PAD - N_NODES))
    src_r = jnp.concatenate([src, pad_idx]).reshape(NUM_WORKERS, CHUNKS, CHUNK)
    dst_r = jnp.concatenate([dst, pad_idx]).reshape(NUM_WORKERS, CHUNKS, CHUNK)
    x_pad = jnp.pad(x, ((0, N_PAD - N_NODES), (0, 0)))
    w2p = jnp.pad(W2, ((0, 0), (0, D_CLS_PAD - D_CLS)))
    b1r = b1.reshape(1, D_HID)
    b2r = jnp.pad(b2, (0, D_CLS_PAD - D_CLS)).reshape(1, D_CLS_PAD)

    h1 = _tc_mm1(x_pad, W1)
    degp = _sc_degree(dst_r)
    dis, g1 = _tc_scale(degp, h1)
    agg1 = _sc_agg32(g1, src_r, dst_r)
    g2 = _tc_layer2(agg1, g1, dis, b1r, w2p)
    agg2 = _sc_agg16(g2, src_r, dst_r)
    return _tc_out(agg2, g2, dis, b2r)
